# baseline (speedup 1.0000x reference)
import functools

import jax
import jax.numpy as jnp
from jax import lax
from jax.experimental import pallas as pl
from jax.experimental.pallas import tpu as pltpu
from jax.experimental.pallas import tpu_sc as plsc

_B, _S, _G = 4, 2048, 4
_VOCAB, _DIM = 100000, 128
_NC, _NS = 2, 16
_NW = _NC * _NS
_ROWS = _B * _S
_RPW = _ROWS // _NW
_GPW = _RPW * _G
_CHUNK = 128
_NCHUNK = _GPW // _CHUNK
_OPC = _CHUNK // _G
_NBUF = 4

_mesh = plsc.VectorSubcoreMesh(core_axis_name="c", subcore_axis_name="s")


@functools.partial(
    pl.kernel,
    mesh=_mesh,
    out_type=jax.ShapeDtypeStruct((_ROWS, _DIM), jnp.float32),
    scratch_types=[
        pltpu.VMEM((_GPW,), jnp.int32),
        pltpu.VMEM((_CHUNK, _DIM), jnp.float32),
        pltpu.VMEM((_CHUNK, _DIM), jnp.float32),
        pltpu.VMEM((_CHUNK, _DIM), jnp.float32),
        pltpu.VMEM((_CHUNK, _DIM), jnp.float32),
        pltpu.VMEM((_OPC, _DIM), jnp.float32),
        pltpu.VMEM((_OPC, _DIM), jnp.float32),
        pltpu.VMEM((_OPC, _DIM), jnp.float32),
        pltpu.VMEM((_OPC, _DIM), jnp.float32),
        pltpu.SemaphoreType.DMA,
        pltpu.SemaphoreType.DMA,
        pltpu.SemaphoreType.DMA,
        pltpu.SemaphoreType.DMA,
        pltpu.SemaphoreType.DMA,
        pltpu.SemaphoreType.DMA,
        pltpu.SemaphoreType.DMA,
        pltpu.SemaphoreType.DMA,
    ],
)
def _embed_sum(x_hbm, tab_hbm, out_hbm,
               idx_v, rows_0, rows_1, rows_2, rows_3,
               outb_0, outb_1, outb_2, outb_3,
               gsem_0, gsem_1, gsem_2, gsem_3,
               osem_0, osem_1, osem_2, osem_3):
    rows = (rows_0, rows_1, rows_2, rows_3)
    outb = (outb_0, outb_1, outb_2, outb_3)
    gsem = (gsem_0, gsem_1, gsem_2, gsem_3)
    osem = (osem_0, osem_1, osem_2, osem_3)

    wid = lax.axis_index("s") * _NC + lax.axis_index("c")
    obase = wid * _RPW
    pltpu.sync_copy(x_hbm.at[pl.ds(wid * _GPW, _GPW)], idx_v)

    off = (lax.iota(jnp.int32, 16) % _G) * _VOCAB
    for i in range(_GPW // 16):
        sl = pl.ds(i * 16, 16)
        idx_v[sl] = idx_v[sl] + off

    def gather(j, p):
        return pltpu.async_copy(
            tab_hbm.at[idx_v.at[pl.ds(j * _CHUNK, _CHUNK)]], rows[p], gsem[p]
        )

    for p in range(_NBUF):
        gather(p, p)

    def group_body(i, carry):
        for p in range(_NBUF):
            j = i * _NBUF + p
            pltpu.make_async_copy(
                tab_hbm.at[idx_v.at[pl.ds(j * _CHUNK, _CHUNK)]], rows[p], gsem[p]
            ).wait()
            @pl.when(i > 0)
            def _():
                pltpu.make_async_copy(
                    outb[p], out_hbm.at[pl.ds(obase, _OPC)], osem[p]
                ).wait()
            buf, ob = rows[p], outb[p]
            for r in range(_OPC):
                for c in range(_DIM // 16):
                    sl = pl.ds(c * 16, 16)
                    ob[r, sl] = (buf[4 * r, sl] + buf[4 * r + 1, sl]) + (
                        buf[4 * r + 2, sl] + buf[4 * r + 3, sl]
                    )
            pltpu.async_copy(ob, out_hbm.at[pl.ds(obase + j * _OPC, _OPC)], osem[p])
            @pl.when(i + 1 < _NCHUNK // _NBUF)
            def _():
                gather(j + _NBUF, p)
        return carry

    lax.fori_loop(0, _NCHUNK // _NBUF, group_body, 0)

    for p in range(_NBUF):
        pltpu.make_async_copy(
            outb[p], out_hbm.at[pl.ds(obase, _OPC)], osem[p]
        ).wait()


def kernel(x, tables):
    xf = x.reshape(_ROWS * _G)
    tf = tables.reshape(_G * _VOCAB, _DIM)
    out = _embed_sum(xf, tf)
    return out.reshape(_B, _S, _DIM)


# speedup vs baseline: 1.6450x; 1.6450x over previous
import functools

import jax
import jax.numpy as jnp
from jax import lax
from jax.experimental import pallas as pl
from jax.experimental.pallas import tpu as pltpu
from jax.experimental.pallas import tpu_sc as plsc

_B, _S, _G = 4, 2048, 4
_VOCAB, _DIM = 100000, 128
_NC, _NS = 2, 16
_NW = _NC * _NS
_ROWS = _B * _S
_RPW = _ROWS // _NW
_GPW = _RPW * _G
_CHUNK = 128
_NCHUNK = _GPW // _CHUNK
_OPC = _CHUNK // _G

_mesh = plsc.VectorSubcoreMesh(core_axis_name="c", subcore_axis_name="s")


@functools.partial(
    pl.kernel,
    mesh=_mesh,
    out_type=jax.ShapeDtypeStruct((_ROWS, _DIM), jnp.float32),
    scratch_types=[
        pltpu.VMEM((_GPW,), jnp.int32),
        pltpu.VMEM((_CHUNK, _DIM), jnp.float32),
        pltpu.VMEM((_CHUNK, _DIM), jnp.float32),
        pltpu.VMEM((_RPW, _DIM), jnp.float32),
        pltpu.SemaphoreType.DMA,
        pltpu.SemaphoreType.DMA,
    ],
)
def _embed_sum(x_hbm, tab_hbm, out_hbm, idx_v, rows_0, rows_1, out_v,
               sem_0, sem_1):
    wid = lax.axis_index("s") * _NC + lax.axis_index("c")
    with jax.named_scope("idx_load"):
        pltpu.sync_copy(x_hbm.at[pl.ds(wid * _GPW, _GPW)], idx_v)

    with jax.named_scope("idx_offset"):
        off = (lax.iota(jnp.int32, 16) % _G) * _VOCAB
        for i in range(_GPW // 16):
            sl = pl.ds(i * 16, 16)
            idx_v[sl] = idx_v[sl] + off

    bufs = (rows_0, rows_1)
    sems = (sem_0, sem_1)

    def start(j):
        return pltpu.async_copy(
            tab_hbm.at[idx_v.at[pl.ds(j * _CHUNK, _CHUNK)]],
            bufs[j % 2],
            sems[j % 2],
        )

    cp = start(0)
    for j in range(_NCHUNK):
        nxt = start(j + 1) if j + 1 < _NCHUNK else None
        with jax.named_scope(f"wait{j}"):
            cp.wait()
        buf = bufs[j % 2]

        with jax.named_scope(f"sum{j}"):
            @plsc.parallel_loop(0, _OPC)
            def _(r, j=j, buf=buf):
                for c in range(_DIM // 16):
                    sl = pl.ds(c * 16, 16)
                    out_v[j * _OPC + r, sl] = (
                        buf[4 * r, sl] + buf[4 * r + 1, sl]
                    ) + (buf[4 * r + 2, sl] + buf[4 * r + 3, sl])

        cp = nxt

    with jax.named_scope("out_store"):
        pltpu.sync_copy(out_v, out_hbm.at[pl.ds(wid * _RPW, _RPW)])


def kernel(x, tables):
    xf = x.reshape(_ROWS * _G)
    tf = tables.reshape(_G * _VOCAB, _DIM)
    out = _embed_sum(xf, tf)
    return out.reshape(_B, _S, _DIM)


# speedup vs baseline: 1.7234x; 1.0476x over previous
import functools

import jax
import jax.numpy as jnp
from jax import lax
from jax.experimental import pallas as pl
from jax.experimental.pallas import tpu as pltpu
from jax.experimental.pallas import tpu_sc as plsc

_B, _S, _G = 4, 2048, 4
_VOCAB, _DIM = 100000, 128
_NC, _NS = 2, 16
_NW = _NC * _NS
_ROWS = _B * _S
_RPW = _ROWS // _NW
_GPW = _RPW * _G
_CHUNK = 128
_NCHUNK = _GPW // _CHUNK
_OPC = _CHUNK // _G
_NBUF = 4

_mesh = plsc.VectorSubcoreMesh(core_axis_name="c", subcore_axis_name="s")


@functools.partial(
    pl.kernel,
    mesh=_mesh,
    out_type=jax.ShapeDtypeStruct((_ROWS, _DIM), jnp.float32),
    scratch_types=[
        pltpu.VMEM((_GPW,), jnp.int32),
        pltpu.VMEM((_CHUNK, _DIM), jnp.float32),
        pltpu.VMEM((_CHUNK, _DIM), jnp.float32),
        pltpu.VMEM((_CHUNK, _DIM), jnp.float32),
        pltpu.VMEM((_CHUNK, _DIM), jnp.float32),
        pltpu.VMEM((_RPW, _DIM), jnp.float32),
        pltpu.SemaphoreType.DMA,
        pltpu.SemaphoreType.DMA,
        pltpu.SemaphoreType.DMA,
        pltpu.SemaphoreType.DMA,
        pltpu.SemaphoreType.DMA,
    ],
)
def _embed_sum(x_hbm, tab_hbm, out_hbm, idx_v, rows_0, rows_1, rows_2, rows_3,
               out_v, sem_0, sem_1, sem_2, sem_3, osem):
    wid = lax.axis_index("s") * _NC + lax.axis_index("c")
    obase = wid * _RPW
    with jax.named_scope("idx_load"):
        pltpu.sync_copy(x_hbm.at[pl.ds(wid * _GPW, _GPW)], idx_v)

    with jax.named_scope("idx_offset"):
        off = (lax.iota(jnp.int32, 16) % _G) * _VOCAB
        for i in range(_GPW // 16):
            sl = pl.ds(i * 16, 16)
            idx_v[sl] = idx_v[sl] + off

    bufs = (rows_0, rows_1, rows_2, rows_3)
    sems = (sem_0, sem_1, sem_2, sem_3)

    def start(j):
        return pltpu.async_copy(
            tab_hbm.at[idx_v.at[pl.ds(j * _CHUNK, _CHUNK)]],
            bufs[j % _NBUF],
            sems[j % _NBUF],
        )

    copies = [start(j) for j in range(_NBUF - 1)]
    ostores = []
    for j in range(_NCHUNK):
        with jax.named_scope(f"wait{j}"):
            copies.pop(0).wait()
        buf = bufs[j % _NBUF]

        with jax.named_scope(f"sum{j}"):
            @plsc.parallel_loop(0, _OPC)
            def _(r, j=j, buf=buf):
                for c in range(_DIM // 16):
                    sl = pl.ds(c * 16, 16)
                    out_v[j * _OPC + r, sl] = (
                        buf[4 * r, sl] + buf[4 * r + 1, sl]
                    ) + (buf[4 * r + 2, sl] + buf[4 * r + 3, sl])

        ostores.append(
            pltpu.async_copy(
                out_v.at[pl.ds(j * _OPC, _OPC)],
                out_hbm.at[pl.ds(obase + j * _OPC, _OPC)],
                osem,
            )
        )
        if j + _NBUF - 1 < _NCHUNK:
            copies.append(start(j + _NBUF - 1))

    with jax.named_scope("out_drain"):
        for c in ostores:
            c.wait()


def kernel(x, tables):
    xf = x.reshape(_ROWS * _G)
    tf = tables.reshape(_G * _VOCAB, _DIM)
    out = _embed_sum(xf, tf)
    return out.reshape(_B, _S, _DIM)
